# baseline (speedup 1.0000x reference)
import functools
import math

import jax
import jax.numpy as jnp
import numpy as np
from jax import lax
from jax.experimental import pallas as pl
from jax.experimental.pallas import tpu as pltpu
from jax.experimental.pallas import tpu_sc as plsc

bf16 = jnp.bfloat16
f32 = jnp.float32
i32 = jnp.int32

B, N, C, KNN, M = 4, 2048, 128, 32, 1024
TA = 256
TC = 128
TD = 256
INTMIN = np.int32(-2147483648)


def _knn_kernel(bofs, p_ref, prow_ref, idx_ref):
    pfull = p_ref[...]
    prow = prow_ref[...]
    aa_full = jnp.sum(pfull * pfull, axis=1)
    aa_row = jnp.sum(prow * prow, axis=1)
    inner = -2.0 * lax.dot_general(
        prow.astype(bf16), pfull.astype(bf16),
        (((1,), (1,)), ((), ())), preferred_element_type=f32)
    d = (-aa_row[:, None]) - inner
    d = d - aa_full[None, :]
    col = lax.broadcasted_iota(i32, (TA, N), 1)
    kcol = lax.broadcasted_iota(i32, (TA, KNN), 1)
    neginf = jnp.float32(-jnp.inf)

    def body(t, carry):
        keys, idxacc = carry
        jstar = jnp.argmax(keys, axis=1).astype(i32)
        keys = jnp.where(col == jstar[:, None], neginf, keys)
        idxacc = jnp.where(kcol == t, jstar[:, None], idxacc)
        return keys, idxacc

    _, idxacc = lax.fori_loop(0, KNN, body, (d, jnp.zeros((TA, KNN), i32)))
    idx_ref[...] = idxacc + bofs


def _pallas_knn_b(pcd_b, bofs):
    return pl.pallas_call(
        functools.partial(_knn_kernel, bofs),
        grid=(N // TA,),
        in_specs=[
            pl.BlockSpec((N, C), lambda r: (0, 0)),
            pl.BlockSpec((TA, C), lambda r: (r, 0)),
        ],
        out_specs=pl.BlockSpec((TA, KNN), lambda r: (r, 0)),
        out_shape=jax.ShapeDtypeStruct((N, KNN), i32),
    )(pcd_b, pcd_b)


def _sc_gather(table, idxg, rows_per_worker, chunks):
    n_out = idxg.shape[0]
    mesh = plsc.VectorSubcoreMesh(core_axis_name="c", subcore_axis_name="s")

    @functools.partial(
        pl.kernel, mesh=mesh,
        out_type=jax.ShapeDtypeStruct((n_out, C), f32),
        scratch_types=[
            pltpu.VMEM((rows_per_worker,), i32),
            pltpu.VMEM((2, 128, C), f32),
            pltpu.SemaphoreType.DMA,
            pltpu.SemaphoreType.DMA,
        ],
    )
    def gath(table_hbm, idx_hbm, out_hbm, idx_v, rows_v, s0, s1):
        nc = 2
        wid = lax.axis_index("s") * nc + lax.axis_index("c")
        base = wid * rows_per_worker
        pltpu.sync_copy(idx_hbm.at[pl.ds(base, rows_per_worker)], idx_v)
        sems = (s0, s1)
        cps = [None, None]
        for g in range(chunks):
            cps[g % 2] = pltpu.async_copy(
                table_hbm.at[idx_v.at[pl.ds(g * 128, 128)]],
                rows_v.at[g % 2], sems[g % 2])
            if g >= 1:
                cps[(g - 1) % 2].wait()
                pltpu.sync_copy(rows_v.at[(g - 1) % 2],
                                out_hbm.at[pl.ds(base + (g - 1) * 128, 128)])
        cps[(chunks - 1) % 2].wait()
        pltpu.sync_copy(rows_v.at[(chunks - 1) % 2],
                        out_hbm.at[pl.ds(base + (chunks - 1) * 128, 128)])

    return gath(table, idxg)


def _attn_kernel(nb_ref, prow_ref, wq_ref, wk_ref, wv_ref,
                 sel_ref, outf_ref):
    nb = nb_ref[...]
    prow = prow_ref[...]
    wq = wq_ref[...].astype(bf16)
    wk = wk_ref[...].astype(bf16)
    wv = wv_ref[...].astype(bf16)

    q = lax.dot_general(prow.astype(bf16), wq,
                        (((1,), (1,)), ((), ())), preferred_element_type=f32)
    diff = nb - prow[:, None, :]
    dflat = diff.reshape(TC * KNN, C).astype(bf16)
    k = lax.dot_general(dflat, wk, (((1,), (1,)), ((), ())),
                        preferred_element_type=f32).reshape(TC, KNN, C)
    v = lax.dot_general(dflat, wv, (((1,), (1,)), ((), ())),
                        preferred_element_type=f32).reshape(TC, KNN, C)

    qb = q.astype(bf16).astype(f32)
    kb = k.astype(bf16).astype(f32)
    energy = jnp.sum(qb[:, None, :] * kb, axis=-1)
    es = energy / math.sqrt(128.0)
    mx = jnp.max(es, axis=-1, keepdims=True)
    w = jnp.exp(es - mx)
    s = jnp.sum(w, axis=-1, keepdims=True)
    att = w / s

    def _tree32(z):
        u = ((z[:, 0:8] + z[:, 8:16]) + z[:, 16:24]) + z[:, 24:32]
        h = u[:, 0:4] + u[:, 4:8]
        h = h[:, 0:2] + h[:, 2:4]
        return h[:, 0:1] + h[:, 1:2]

    mean = _tree32(att) / 32.0
    cc = att - mean
    var = _tree32(cc * cc)[:, 0] / 32.0
    sel_ref[0] = jnp.sqrt(var).reshape(8, TC // 8)

    attb = att.astype(bf16).astype(f32)
    vb = v.astype(bf16).astype(f32)
    outf_ref[...] = jnp.sum(attb[:, :, None] * vb, axis=1)


def _pallas_attn_b(neighbors_b, pcd_b, Wq, Wk, Wv):
    return pl.pallas_call(
        _attn_kernel,
        grid=(N // TC,),
        in_specs=[
            pl.BlockSpec((TC, KNN, C), lambda r: (r, 0, 0)),
            pl.BlockSpec((TC, C), lambda r: (r, 0)),
            pl.BlockSpec((C, C), lambda r: (0, 0)),
            pl.BlockSpec((C, C), lambda r: (0, 0)),
            pl.BlockSpec((C, C), lambda r: (0, 0)),
        ],
        out_specs=[
            pl.BlockSpec((1, 8, TC // 8), lambda r: (r, 0, 0)),
            pl.BlockSpec((TC, C), lambda r: (r, 0)),
        ],
        out_shape=[
            jax.ShapeDtypeStruct((N // TC, 8, TC // 8), f32),
            jax.ShapeDtypeStruct((N, C), f32),
        ],
    )(neighbors_b, pcd_b, Wq, Wk, Wv)


def _rank_kernel(sel_ref, tile_ref, rank_ref):
    full = sel_ref[0, 0]
    tile = tile_ref[0, 0]
    r = pl.program_id(1)
    colv = lax.broadcasted_iota(i32, (TD, N), 1)
    rowg = r * TD + lax.broadcasted_iota(i32, (TD, N), 0)
    fb = full[None, :]
    tb = tile[:, None]
    gt = (fb > tb).astype(i32)
    eqlow = ((fb == tb) & (colv < rowg)).astype(i32)
    rank = jnp.sum(gt + eqlow, axis=1)
    rank_ref[0, 0] = rank.reshape(8, TD // 8)


def _pallas_rank(sel3):
    return pl.pallas_call(
        _rank_kernel,
        grid=(B, N // TD),
        in_specs=[
            pl.BlockSpec((1, 1, N), lambda b, r: (b, 0, 0)),
            pl.BlockSpec((1, 1, TD), lambda b, r: (b, 0, r)),
        ],
        out_specs=pl.BlockSpec((1, 1, 8, TD // 8), lambda b, r: (b, r, 0, 0)),
        out_shape=jax.ShapeDtypeStruct((B, N // TD, 8, TD // 8), i32),
    )(sel3, sel3)


def _invert_kernel(rank_ref, selidx_ref):
    full = rank_ref[0, 0]
    b = pl.program_id(0)
    r = pl.program_id(1)
    colv = lax.broadcasted_iota(i32, (TD, N), 1)
    mvals = r * TD + lax.broadcasted_iota(i32, (TD, N), 0)
    eq = full[None, :] == mvals
    src = jnp.sum(jnp.where(eq, colv, 0), axis=1)
    selidx_ref[0, 0] = (src + b * N).reshape(8, TD // 8)


def _pallas_invert(rank3):
    return pl.pallas_call(
        _invert_kernel,
        grid=(B, M // TD),
        in_specs=[pl.BlockSpec((1, 1, N), lambda b, r: (b, 0, 0))],
        out_specs=pl.BlockSpec((1, 1, 8, TD // 8), lambda b, r: (b, r, 0, 0)),
        out_shape=jax.ShapeDtypeStruct((B, M // TD, 8, TD // 8), i32),
    )(rank3)


def kernel(x, Wq, Wk, Wv):
    pcd = jnp.transpose(x, (0, 2, 1))
    pflat = pcd.reshape(B * N, C)

    sels, outs = [], []
    for b in range(B):
        pcd_b = pcd[b]
        idx_b = _pallas_knn_b(pcd_b, b * N)
        nb_b = _sc_gather(pflat, idx_b.reshape(N * KNN),
                          rows_per_worker=N * KNN // 32,
                          chunks=N * KNN // 32 // 128)
        sel_b, out_b = _pallas_attn_b(nb_b.reshape(N, KNN, C), pcd_b,
                                      Wq, Wk, Wv)
        sels.append(sel_b.reshape(N))
        outs.append(out_b)

    sel3 = jnp.stack(sels).reshape(B, 1, N)
    outfull = jnp.stack(outs)

    rank3 = _pallas_rank(sel3).reshape(B, 1, N)
    selidx = _pallas_invert(rank3).reshape(B * M)

    outrows = _sc_gather(outfull.reshape(B * N, C), selidx,
                         rows_per_worker=B * M // 32,
                         chunks=B * M // 32 // 128)
    return jnp.transpose(outrows.reshape(B, M, C), (0, 2, 1))


# speedup vs baseline: 1.0794x; 1.0794x over previous
import functools
import math

import jax
import jax.numpy as jnp
import numpy as np
from jax import lax
from jax.experimental import pallas as pl
from jax.experimental.pallas import tpu as pltpu
from jax.experimental.pallas import tpu_sc as plsc

bf16 = jnp.bfloat16
f32 = jnp.float32
i32 = jnp.int32

B, N, C, KNN, M = 4, 2048, 128, 32, 1024
TA = 512
TC = 128
TD = 256
INTMIN = np.int32(-2147483648)


def _knn_kernel(bofs, p_ref, prow_ref, idx_ref):
    pfull = p_ref[...]
    prow = prow_ref[...]
    aa_full = jnp.sum(pfull * pfull, axis=1)
    aa_row = jnp.sum(prow * prow, axis=1)
    inner = -2.0 * lax.dot_general(
        prow.astype(bf16), pfull.astype(bf16),
        (((1,), (1,)), ((), ())), preferred_element_type=f32)
    d = (-aa_row[:, None]) - inner
    d = d - aa_full[None, :]
    col = lax.broadcasted_iota(i32, (TA, N), 1)
    kcol = lax.broadcasted_iota(i32, (TA, KNN), 1)
    neginf = jnp.float32(-jnp.inf)

    def body(t, carry):
        keys, idxacc = carry
        jstar = jnp.argmax(keys, axis=1).astype(i32)
        keys = jnp.where(col == jstar[:, None], neginf, keys)
        idxacc = jnp.where(kcol == t, jstar[:, None], idxacc)
        return keys, idxacc

    _, idxacc = lax.fori_loop(0, KNN, body, (d, jnp.zeros((TA, KNN), i32)))
    idx_ref[...] = idxacc + bofs


def _pallas_knn_b(pcd_b, bofs):
    return pl.pallas_call(
        functools.partial(_knn_kernel, bofs),
        grid=(N // TA,),
        in_specs=[
            pl.BlockSpec((N, C), lambda r: (0, 0)),
            pl.BlockSpec((TA, C), lambda r: (r, 0)),
        ],
        out_specs=pl.BlockSpec((TA, KNN), lambda r: (r, 0)),
        out_shape=jax.ShapeDtypeStruct((N, KNN), i32),
    )(pcd_b, pcd_b)


def _sc_gather(table, idxg, rows_per_worker, chunks):
    n_out = idxg.shape[0]
    mesh = plsc.VectorSubcoreMesh(core_axis_name="c", subcore_axis_name="s")

    @functools.partial(
        pl.kernel, mesh=mesh,
        out_type=jax.ShapeDtypeStruct((n_out, C), f32),
        scratch_types=[
            pltpu.VMEM((rows_per_worker,), i32),
            pltpu.VMEM((2, 128, C), f32),
            pltpu.SemaphoreType.DMA,
            pltpu.SemaphoreType.DMA,
        ],
    )
    def gath(table_hbm, idx_hbm, out_hbm, idx_v, rows_v, s0, s1):
        nc = 2
        wid = lax.axis_index("s") * nc + lax.axis_index("c")
        base = wid * rows_per_worker
        pltpu.sync_copy(idx_hbm.at[pl.ds(base, rows_per_worker)], idx_v)
        sems = (s0, s1)
        cps = [None, None]
        for g in range(chunks):
            cps[g % 2] = pltpu.async_copy(
                table_hbm.at[idx_v.at[pl.ds(g * 128, 128)]],
                rows_v.at[g % 2], sems[g % 2])
            if g >= 1:
                cps[(g - 1) % 2].wait()
                pltpu.sync_copy(rows_v.at[(g - 1) % 2],
                                out_hbm.at[pl.ds(base + (g - 1) * 128, 128)])
        cps[(chunks - 1) % 2].wait()
        pltpu.sync_copy(rows_v.at[(chunks - 1) % 2],
                        out_hbm.at[pl.ds(base + (chunks - 1) * 128, 128)])

    return gath(table, idxg)


def _attn_kernel(nb_ref, prow_ref, wq_ref, wk_ref, wv_ref,
                 sel_ref, outf_ref):
    nb = nb_ref[...]
    prow = prow_ref[...]
    wq = wq_ref[...].astype(bf16)
    wk = wk_ref[...].astype(bf16)
    wv = wv_ref[...].astype(bf16)

    q = lax.dot_general(prow.astype(bf16), wq,
                        (((1,), (1,)), ((), ())), preferred_element_type=f32)
    diff = nb - prow[:, None, :]
    dflat = diff.reshape(TC * KNN, C).astype(bf16)
    k = lax.dot_general(dflat, wk, (((1,), (1,)), ((), ())),
                        preferred_element_type=f32).reshape(TC, KNN, C)
    v = lax.dot_general(dflat, wv, (((1,), (1,)), ((), ())),
                        preferred_element_type=f32).reshape(TC, KNN, C)

    qb = q.astype(bf16).astype(f32)
    kb = k.astype(bf16).astype(f32)
    energy = jnp.sum(qb[:, None, :] * kb, axis=-1)
    es = energy / math.sqrt(128.0)
    mx = jnp.max(es, axis=-1, keepdims=True)
    w = jnp.exp(es - mx)
    s = jnp.sum(w, axis=-1, keepdims=True)
    att = w / s

    def _tree32(z):
        u = ((z[:, 0:8] + z[:, 8:16]) + z[:, 16:24]) + z[:, 24:32]
        h = u[:, 0:4] + u[:, 4:8]
        h = h[:, 0:2] + h[:, 2:4]
        return h[:, 0:1] + h[:, 1:2]

    mean = _tree32(att) / 32.0
    cc = att - mean
    var = _tree32(cc * cc)[:, 0] / 32.0
    sel_ref[0] = jnp.sqrt(var).reshape(8, TC // 8)

    attb = att.astype(bf16).astype(f32)
    vb = v.astype(bf16).astype(f32)
    outf_ref[...] = jnp.sum(attb[:, :, None] * vb, axis=1)


def _pallas_attn_b(neighbors_b, pcd_b, Wq, Wk, Wv):
    return pl.pallas_call(
        _attn_kernel,
        grid=(N // TC,),
        in_specs=[
            pl.BlockSpec((TC, KNN, C), lambda r: (r, 0, 0)),
            pl.BlockSpec((TC, C), lambda r: (r, 0)),
            pl.BlockSpec((C, C), lambda r: (0, 0)),
            pl.BlockSpec((C, C), lambda r: (0, 0)),
            pl.BlockSpec((C, C), lambda r: (0, 0)),
        ],
        out_specs=[
            pl.BlockSpec((1, 8, TC // 8), lambda r: (r, 0, 0)),
            pl.BlockSpec((TC, C), lambda r: (r, 0)),
        ],
        out_shape=[
            jax.ShapeDtypeStruct((N // TC, 8, TC // 8), f32),
            jax.ShapeDtypeStruct((N, C), f32),
        ],
    )(neighbors_b, pcd_b, Wq, Wk, Wv)


def _rank_kernel(sel_ref, tile_ref, rank_ref):
    full = sel_ref[0, 0]
    tile = tile_ref[0, 0]
    r = pl.program_id(1)
    colv = lax.broadcasted_iota(i32, (TD, N), 1)
    rowg = r * TD + lax.broadcasted_iota(i32, (TD, N), 0)
    fb = full[None, :]
    tb = tile[:, None]
    gt = (fb > tb).astype(i32)
    eqlow = ((fb == tb) & (colv < rowg)).astype(i32)
    rank = jnp.sum(gt + eqlow, axis=1)
    rank_ref[0, 0] = rank.reshape(8, TD // 8)


def _pallas_rank(sel3):
    return pl.pallas_call(
        _rank_kernel,
        grid=(B, N // TD),
        in_specs=[
            pl.BlockSpec((1, 1, N), lambda b, r: (b, 0, 0)),
            pl.BlockSpec((1, 1, TD), lambda b, r: (b, 0, r)),
        ],
        out_specs=pl.BlockSpec((1, 1, 8, TD // 8), lambda b, r: (b, r, 0, 0)),
        out_shape=jax.ShapeDtypeStruct((B, N // TD, 8, TD // 8), i32),
    )(sel3, sel3)


def _invert_kernel(rank_ref, selidx_ref):
    full = rank_ref[0, 0]
    b = pl.program_id(0)
    r = pl.program_id(1)
    colv = lax.broadcasted_iota(i32, (TD, N), 1)
    mvals = r * TD + lax.broadcasted_iota(i32, (TD, N), 0)
    eq = full[None, :] == mvals
    src = jnp.sum(jnp.where(eq, colv, 0), axis=1)
    selidx_ref[0, 0] = (src + b * N).reshape(8, TD // 8)


def _pallas_invert(rank3):
    return pl.pallas_call(
        _invert_kernel,
        grid=(B, M // TD),
        in_specs=[pl.BlockSpec((1, 1, N), lambda b, r: (b, 0, 0))],
        out_specs=pl.BlockSpec((1, 1, 8, TD // 8), lambda b, r: (b, r, 0, 0)),
        out_shape=jax.ShapeDtypeStruct((B, M // TD, 8, TD // 8), i32),
    )(rank3)


def kernel(x, Wq, Wk, Wv):
    pcd = jnp.transpose(x, (0, 2, 1))
    pflat = pcd.reshape(B * N, C)

    sels, outs = [], []
    for b in range(B):
        pcd_b = pcd[b]
        idx_b = _pallas_knn_b(pcd_b, b * N)
        nb_b = _sc_gather(pflat, idx_b.reshape(N * KNN),
                          rows_per_worker=N * KNN // 32,
                          chunks=N * KNN // 32 // 128)
        sel_b, out_b = _pallas_attn_b(nb_b.reshape(N, KNN, C), pcd_b,
                                      Wq, Wk, Wv)
        sels.append(sel_b.reshape(N))
        outs.append(out_b)

    sel3 = jnp.stack(sels).reshape(B, 1, N)
    outfull = jnp.stack(outs)

    rank3 = _pallas_rank(sel3).reshape(B, 1, N)
    selidx = _pallas_invert(rank3).reshape(B * M)

    outrows = _sc_gather(outfull.reshape(B * N, C), selidx,
                         rows_per_worker=B * M // 32,
                         chunks=B * M // 32 // 128)
    return jnp.transpose(outrows.reshape(B, M, C), (0, 2, 1))


# speedup vs baseline: 1.1213x; 1.0388x over previous
import functools
import math

import jax
import jax.numpy as jnp
import numpy as np
from jax import lax
from jax.experimental import pallas as pl
from jax.experimental.pallas import tpu as pltpu
from jax.experimental.pallas import tpu_sc as plsc

bf16 = jnp.bfloat16
f32 = jnp.float32
i32 = jnp.int32

B, N, C, KNN, M = 4, 2048, 128, 32, 1024
TA = 1024
TC = 128
TD = 256
INTMIN = np.int32(-2147483648)


def _knn_kernel(bofs, p_ref, prow_ref, idx_ref):
    pfull = p_ref[...]
    prow = prow_ref[...]
    aa_full = jnp.sum(pfull * pfull, axis=1)
    aa_row = jnp.sum(prow * prow, axis=1)
    inner = -2.0 * lax.dot_general(
        prow.astype(bf16), pfull.astype(bf16),
        (((1,), (1,)), ((), ())), preferred_element_type=f32)
    d = (-aa_row[:, None]) - inner
    d = d - aa_full[None, :]
    col = lax.broadcasted_iota(i32, (TA, N), 1)
    kcol = lax.broadcasted_iota(i32, (TA, KNN), 1)
    neginf = jnp.float32(-jnp.inf)

    def body(t, carry):
        keys, idxacc = carry
        jstar = jnp.argmax(keys, axis=1).astype(i32)
        keys = jnp.where(col == jstar[:, None], neginf, keys)
        idxacc = jnp.where(kcol == t, jstar[:, None], idxacc)
        return keys, idxacc

    _, idxacc = lax.fori_loop(0, KNN, body, (d, jnp.zeros((TA, KNN), i32)))
    idx_ref[...] = idxacc + bofs


def _pallas_knn_b(pcd_b, bofs):
    return pl.pallas_call(
        functools.partial(_knn_kernel, bofs),
        grid=(N // TA,),
        in_specs=[
            pl.BlockSpec((N, C), lambda r: (0, 0)),
            pl.BlockSpec((TA, C), lambda r: (r, 0)),
        ],
        out_specs=pl.BlockSpec((TA, KNN), lambda r: (r, 0)),
        out_shape=jax.ShapeDtypeStruct((N, KNN), i32),
    )(pcd_b, pcd_b)


def _sc_gather(table, idxg, rows_per_worker, chunks):
    n_out = idxg.shape[0]
    mesh = plsc.VectorSubcoreMesh(core_axis_name="c", subcore_axis_name="s")

    @functools.partial(
        pl.kernel, mesh=mesh,
        out_type=jax.ShapeDtypeStruct((n_out, C), f32),
        scratch_types=[
            pltpu.VMEM((rows_per_worker,), i32),
            pltpu.VMEM((2, 128, C), f32),
            pltpu.SemaphoreType.DMA,
            pltpu.SemaphoreType.DMA,
        ],
    )
    def gath(table_hbm, idx_hbm, out_hbm, idx_v, rows_v, s0, s1):
        nc = 2
        wid = lax.axis_index("s") * nc + lax.axis_index("c")
        base = wid * rows_per_worker
        pltpu.sync_copy(idx_hbm.at[pl.ds(base, rows_per_worker)], idx_v)
        sems = (s0, s1)
        cps = [None, None]
        for g in range(chunks):
            cps[g % 2] = pltpu.async_copy(
                table_hbm.at[idx_v.at[pl.ds(g * 128, 128)]],
                rows_v.at[g % 2], sems[g % 2])
            if g >= 1:
                cps[(g - 1) % 2].wait()
                pltpu.sync_copy(rows_v.at[(g - 1) % 2],
                                out_hbm.at[pl.ds(base + (g - 1) * 128, 128)])
        cps[(chunks - 1) % 2].wait()
        pltpu.sync_copy(rows_v.at[(chunks - 1) % 2],
                        out_hbm.at[pl.ds(base + (chunks - 1) * 128, 128)])

    return gath(table, idxg)


def _attn_kernel(nb_ref, prow_ref, wq_ref, wk_ref, wv_ref,
                 sel_ref, outf_ref):
    nb = nb_ref[...]
    prow = prow_ref[...]
    wq = wq_ref[...].astype(bf16)
    wk = wk_ref[...].astype(bf16)
    wv = wv_ref[...].astype(bf16)

    q = lax.dot_general(prow.astype(bf16), wq,
                        (((1,), (1,)), ((), ())), preferred_element_type=f32)
    diff = nb - prow[:, None, :]
    dflat = diff.reshape(TC * KNN, C).astype(bf16)
    k = lax.dot_general(dflat, wk, (((1,), (1,)), ((), ())),
                        preferred_element_type=f32).reshape(TC, KNN, C)
    v = lax.dot_general(dflat, wv, (((1,), (1,)), ((), ())),
                        preferred_element_type=f32).reshape(TC, KNN, C)

    qb = q.astype(bf16).astype(f32)
    kb = k.astype(bf16).astype(f32)
    energy = jnp.sum(qb[:, None, :] * kb, axis=-1)
    es = energy / math.sqrt(128.0)
    mx = jnp.max(es, axis=-1, keepdims=True)
    w = jnp.exp(es - mx)
    s = jnp.sum(w, axis=-1, keepdims=True)
    att = w / s

    def _tree32(z):
        u = ((z[:, 0:8] + z[:, 8:16]) + z[:, 16:24]) + z[:, 24:32]
        h = u[:, 0:4] + u[:, 4:8]
        h = h[:, 0:2] + h[:, 2:4]
        return h[:, 0:1] + h[:, 1:2]

    mean = _tree32(att) / 32.0
    cc = att - mean
    var = _tree32(cc * cc)[:, 0] / 32.0
    sel_ref[0] = jnp.sqrt(var).reshape(8, TC // 8)

    attb = att.astype(bf16).astype(f32)
    vb = v.astype(bf16).astype(f32)
    outf_ref[...] = jnp.sum(attb[:, :, None] * vb, axis=1)


def _pallas_attn_b(neighbors_b, pcd_b, Wq, Wk, Wv):
    return pl.pallas_call(
        _attn_kernel,
        grid=(N // TC,),
        in_specs=[
            pl.BlockSpec((TC, KNN, C), lambda r: (r, 0, 0)),
            pl.BlockSpec((TC, C), lambda r: (r, 0)),
            pl.BlockSpec((C, C), lambda r: (0, 0)),
            pl.BlockSpec((C, C), lambda r: (0, 0)),
            pl.BlockSpec((C, C), lambda r: (0, 0)),
        ],
        out_specs=[
            pl.BlockSpec((1, 8, TC // 8), lambda r: (r, 0, 0)),
            pl.BlockSpec((TC, C), lambda r: (r, 0)),
        ],
        out_shape=[
            jax.ShapeDtypeStruct((N // TC, 8, TC // 8), f32),
            jax.ShapeDtypeStruct((N, C), f32),
        ],
    )(neighbors_b, pcd_b, Wq, Wk, Wv)


def _rank_kernel(sel_ref, tile_ref, rank_ref):
    full = sel_ref[0, 0]
    tile = tile_ref[0, 0]
    r = pl.program_id(1)
    colv = lax.broadcasted_iota(i32, (TD, N), 1)
    rowg = r * TD + lax.broadcasted_iota(i32, (TD, N), 0)
    fb = full[None, :]
    tb = tile[:, None]
    gt = (fb > tb).astype(i32)
    eqlow = ((fb == tb) & (colv < rowg)).astype(i32)
    rank = jnp.sum(gt + eqlow, axis=1)
    rank_ref[0, 0] = rank.reshape(8, TD // 8)


def _pallas_rank(sel3):
    return pl.pallas_call(
        _rank_kernel,
        grid=(B, N // TD),
        in_specs=[
            pl.BlockSpec((1, 1, N), lambda b, r: (b, 0, 0)),
            pl.BlockSpec((1, 1, TD), lambda b, r: (b, 0, r)),
        ],
        out_specs=pl.BlockSpec((1, 1, 8, TD // 8), lambda b, r: (b, r, 0, 0)),
        out_shape=jax.ShapeDtypeStruct((B, N // TD, 8, TD // 8), i32),
    )(sel3, sel3)


def _invert_kernel(rank_ref, selidx_ref):
    full = rank_ref[0, 0]
    b = pl.program_id(0)
    r = pl.program_id(1)
    colv = lax.broadcasted_iota(i32, (TD, N), 1)
    mvals = r * TD + lax.broadcasted_iota(i32, (TD, N), 0)
    eq = full[None, :] == mvals
    src = jnp.sum(jnp.where(eq, colv, 0), axis=1)
    selidx_ref[0, 0] = (src + b * N).reshape(8, TD // 8)


def _pallas_invert(rank3):
    return pl.pallas_call(
        _invert_kernel,
        grid=(B, M // TD),
        in_specs=[pl.BlockSpec((1, 1, N), lambda b, r: (b, 0, 0))],
        out_specs=pl.BlockSpec((1, 1, 8, TD // 8), lambda b, r: (b, r, 0, 0)),
        out_shape=jax.ShapeDtypeStruct((B, M // TD, 8, TD // 8), i32),
    )(rank3)


def kernel(x, Wq, Wk, Wv):
    pcd = jnp.transpose(x, (0, 2, 1))
    pflat = pcd.reshape(B * N, C)

    sels, outs = [], []
    for b in range(B):
        pcd_b = pcd[b]
        idx_b = _pallas_knn_b(pcd_b, b * N)
        nb_b = _sc_gather(pflat, idx_b.reshape(N * KNN),
                          rows_per_worker=N * KNN // 32,
                          chunks=N * KNN // 32 // 128)
        sel_b, out_b = _pallas_attn_b(nb_b.reshape(N, KNN, C), pcd_b,
                                      Wq, Wk, Wv)
        sels.append(sel_b.reshape(N))
        outs.append(out_b)

    sel3 = jnp.stack(sels).reshape(B, 1, N)
    outfull = jnp.stack(outs)

    rank3 = _pallas_rank(sel3).reshape(B, 1, N)
    selidx = _pallas_invert(rank3).reshape(B * M)

    outrows = _sc_gather(outfull.reshape(B * N, C), selidx,
                         rows_per_worker=B * M // 32,
                         chunks=B * M // 32 // 128)
    return jnp.transpose(outrows.reshape(B, M, C), (0, 2, 1))
